# baseline (speedup 1.0000x reference)
import functools

import jax
import jax.numpy as jnp
from jax import lax
from jax.experimental import pallas as pl
from jax.experimental.pallas import tpu as pltpu
from jax.experimental.pallas import tpu_sc as plsc

_L = 16
_NC = 2
_NS = 16
_NW = _NC * _NS
_CHUNK = 128
_NCATE = 3


def _maskout_body(bpw, x_hbm, label_hbm, out_hbm,
                  label_v, idxf_v, idx2_v, buf_v, cate_s, sem_g, sem_s):
    n_chunks = bpw // _CHUNK + _NCATE - 1
    flat = n_chunks * _CHUNK
    cid = lax.axis_index("c")
    sid = lax.axis_index("s")
    wid = sid * _NC + cid
    base = wid * bpw
    lane = lax.iota(jnp.int32, _L)

    pltpu.sync_copy(label_hbm.at[pl.ds(base, bpw)], label_v.at[pl.ds(0, bpw)])

    cstar = label_v[pl.ds(0, _L)][0]
    base_splat = jnp.full((_L,), 0, jnp.int32) + base
    for j in range(flat // _L):
        idxf_v[pl.ds(j * _L, _L)] = base_splat
    for t in range(n_chunks):
        cate_s[t] = cstar

    chunks_used = base - base
    for c in range(_NCATE):
        start = chunks_used * _CHUNK
        off = start
        for j in range(bpw // _L):
            lbl = label_v[pl.ds(j * _L, _L)]
            ids = base + j * _L + lane
            m = lbl == c
            mi = m.astype(jnp.int32)
            pos = off + plsc.cumsum(mi) - mi
            plsc.store_scatter(idxf_v, [pos], ids, mask=m)
            off = off + jnp.sum(mi)
        n_c = off - start

        @pl.when(n_c % _CHUNK != 0)
        def _pad(start=start, off=off, n_c=n_c):
            first = idxf_v[pl.ds(start, _L)][0]
            first_splat = jnp.full((_L,), 0, jnp.int32) + first
            t0 = start + (n_c // _L) * _L
            tail = idxf_v[pl.ds(t0, _L)]
            keep = lane < (off - t0)
            idxf_v[pl.ds(t0, _L)] = jnp.where(keep, tail, first_splat)
            end = start + ((n_c + _CHUNK - 1) // _CHUNK) * _CHUNK

            def fill(u, _):
                idxf_v[pl.ds(t0 + _L + u * _L, _L)] = first_splat
                return 0

            lax.fori_loop(0, (end - (t0 + _L)) // _L, fill, 0)

        nch_c = (n_c + _CHUNK - 1) // _CHUNK

        def wcate(t, _, c=c):
            cate_s[t] = c
            return 0

        lax.fori_loop(chunks_used, chunks_used + nch_c, wcate, 0)
        chunks_used = chunks_used + nch_c

    for t in range(n_chunks):
        for j in range(_CHUNK // _L):
            idx2_v[t, pl.ds(j * _L, _L)] = idxf_v[pl.ds(t * _CHUNK + j * _L, _L)]

    gathers = []
    for t in range(n_chunks):
        c_t = cate_s[t]
        gathers.append(
            pltpu.async_copy(
                x_hbm.at[:, c_t].at[idx2_v.at[t]], buf_v.at[t], sem_g.at[t]
            )
        )
    scatters = []
    for t in range(n_chunks):
        gathers[t].wait()
        scatters.append(
            pltpu.async_copy(buf_v.at[t], out_hbm.at[idx2_v.at[t]], sem_s.at[t])
        )
    for t in range(n_chunks):
        scatters[t].wait()


@jax.jit
def kernel(x, label):
    batch, nr_cate, d = x.shape
    bpw = batch // _NW
    n_chunks = bpw // _CHUNK + _NCATE - 1
    flat = n_chunks * _CHUNK

    mesh = plsc.VectorSubcoreMesh(core_axis_name="c", subcore_axis_name="s")
    run = pl.kernel(
        functools.partial(_maskout_body, bpw),
        out_type=jax.ShapeDtypeStruct((batch, d), x.dtype),
        mesh=mesh,
        scratch_types=[
            pltpu.VMEM((bpw + _L,), jnp.int32),
            pltpu.VMEM((flat,), jnp.int32),
            pltpu.VMEM((n_chunks, _CHUNK), jnp.int32),
            pltpu.VMEM((n_chunks, _CHUNK, d), jnp.float32),
            pltpu.SMEM((8,), jnp.int32),
            pltpu.SemaphoreType.DMA((n_chunks,)),
            pltpu.SemaphoreType.DMA((n_chunks,)),
        ],
        compiler_params=pltpu.CompilerParams(needs_layout_passes=False),
    )
    return run(x, label)
